# baseline (speedup 1.0000x reference)
import functools

import jax
import jax.numpy as jnp
import numpy as np
from jax import lax
from jax.experimental import pallas as pl
from jax.experimental.pallas import tpu as pltpu
from jax.experimental.pallas import tpu_sc as plsc

_N = 10000
_NOISE_DIM = 16
try:
    _NOISE = np.asarray(jax.random.normal(jax.random.key(42),
                                          (_N, _NOISE_DIM), jnp.float32))
except Exception:
    _NOISE = None


def _prelu(x, a):
    return jnp.where(x > 0, x, a * x)


def _adj_pass_kernel(x_ref, b_ref, a_ref, adj_ref, *rest, cast_out, next_w,
                     emb_out, out_dtype):
    pos = 0
    w_ref = None
    if next_w:
        w_ref = rest[pos]
        pos += 1
    outs = list(rest[pos:])
    o_ref = outs.pop(0)
    h_ref = outs.pop(0) if emb_out else None
    cast_ref = outs.pop(0) if cast_out else None
    xn_ref = outs.pop(0) if next_w else None

    kdim = x_ref.shape[0]
    if cast_out:
        tile = adj_ref[...].astype(jnp.bfloat16)
        npad = cast_ref.shape[1] - kdim
        if npad:
            tile = jnp.concatenate(
                [tile, jnp.zeros((tile.shape[0], npad), jnp.bfloat16)], axis=1)
        cast_ref[...] = tile
        lhs = tile[:, :kdim]
    else:
        lhs = adj_ref[...][:, :kdim]
    acc = jnp.dot(lhs, x_ref[...], preferred_element_type=jnp.float32)
    h = _prelu(acc + b_ref[...], a_ref[0])

    if emb_out:
        half = h.shape[1] // 2
        o_ref[0, :, :] = h[:, half:]
        o_ref[1, :, :] = h[:, :half]
        h_ref[...] = h
    else:
        o_ref[...] = h.astype(out_dtype)

    if next_w:
        hw = h
        if emb_out or w_ref.shape[0] != h.shape[1]:
            hw = h[:, h.shape[1] - w_ref.shape[0]:]
        xn_ref[...] = jnp.dot(hw.astype(jnp.bfloat16), w_ref[...],
                              preferred_element_type=jnp.float32
                              ).astype(jnp.bfloat16)


def _adj_pass(adj, x, b, a, cast_out=False, next_w=None, emb_out=False,
              bm=400, out_dtype=jnp.float32):
    n = adj.shape[0]
    ncols = adj.shape[1]
    n_pad = n
    out_f = x.shape[1]
    grid = (n // bm,)
    b2 = jnp.broadcast_to(b, (1, out_f))
    a2 = jnp.reshape(a, (1,))
    kern = functools.partial(_adj_pass_kernel, cast_out=cast_out,
                             next_w=next_w is not None, emb_out=emb_out,
                             out_dtype=out_dtype)
    in_specs = [
        pl.BlockSpec((n, out_f), lambda i: (0, 0)),
        pl.BlockSpec((1, out_f), lambda i: (0, 0)),
        pl.BlockSpec(memory_space=pltpu.SMEM),
        pl.BlockSpec((bm, ncols), lambda i: (i, 0)),
    ]
    operands = [x, b2, a2, adj]
    out_specs = []
    out_shape = []
    if emb_out:
        half = out_f // 2
        out_specs.append(pl.BlockSpec((2, bm, half), lambda i: (0, i, 0)))
        out_shape.append(jax.ShapeDtypeStruct((2, n, half), jnp.float32))
        out_specs.append(pl.BlockSpec((bm, out_f), lambda i: (i, 0)))
        out_shape.append(jax.ShapeDtypeStruct((n, out_f), jnp.float32))
    else:
        out_specs.append(pl.BlockSpec((bm, out_f), lambda i: (i, 0)))
        out_shape.append(jax.ShapeDtypeStruct((n, out_f), out_dtype))
    if cast_out:
        out_specs.append(pl.BlockSpec((bm, n_pad), lambda i: (i, 0)))
        out_shape.append(jax.ShapeDtypeStruct((n, n_pad), jnp.bfloat16))
    if next_w is not None:
        in_specs.append(pl.BlockSpec(next_w.shape, lambda i: (0, 0)))
        operands.append(next_w)
        nf = next_w.shape[1]
        out_specs.append(pl.BlockSpec((bm, nf), lambda i: (i, 0)))
        out_shape.append(jax.ShapeDtypeStruct((n, nf), jnp.bfloat16))
    res = pl.pallas_call(
        kern,
        grid=grid,
        in_specs=in_specs,
        out_specs=out_specs,
        out_shape=out_shape,
        compiler_params=pltpu.CompilerParams(
            dimension_semantics=("arbitrary",)),
    )(*operands)
    return res


def _xf1_kernel(seq_ref, noise_ref, wg1_ref, bg1_ref, wg2_ref, bg2_ref,
                we1_ref, x_ref):
    g = jax.nn.relu(jnp.dot(noise_ref[...], wg1_ref[...].T,
                            preferred_element_type=jnp.float32) + bg1_ref[...])
    x_gen = jnp.dot(g, wg2_ref[...].T,
                    preferred_element_type=jnp.float32) + bg2_ref[...]
    we1t = we1_ref[...].T
    xg = jnp.dot(x_gen, we1t, preferred_element_type=jnp.float32)
    xs = jnp.dot(seq_ref[...], we1t, preferred_element_type=jnp.float32)
    x_ref[...] = jnp.concatenate([xg, xs], axis=1).astype(jnp.bfloat16)


def _xf1(seq1, noise, Wg1, bg1, Wg2, bg2, We1):
    n = seq1.shape[0]
    nh = We1.shape[0]
    return pl.pallas_call(
        _xf1_kernel,
        out_shape=jax.ShapeDtypeStruct((n, 2 * nh), jnp.bfloat16),
    )(seq1, noise, Wg1, jnp.broadcast_to(bg1, (1, bg1.shape[0])),
      Wg2, jnp.broadcast_to(bg2, (1, bg2.shape[0])), We1)


def _sc_gather(table, idx):
    n, d = table.shape
    k_pad = idx.shape[0]
    info = plsc.get_sparse_core_info()
    nw = info.num_cores * info.num_subcores
    b_per_w = k_pad // nw
    itemsize = jnp.dtype(table.dtype).itemsize
    rows_chunk = b_per_w
    while rows_chunk > 8 and rows_chunk * d * itemsize > 330_000:
        rows_chunk //= 2
    n_chunks = b_per_w // rows_chunk
    mesh = plsc.VectorSubcoreMesh(core_axis_name="c", subcore_axis_name="s")

    @functools.partial(
        pl.kernel, mesh=mesh,
        out_type=jax.ShapeDtypeStruct((k_pad, d), table.dtype),
        scratch_types=[
            pltpu.VMEM((rows_chunk,), jnp.int32),
            pltpu.VMEM((rows_chunk, d), table.dtype),
            pltpu.SemaphoreType.DMA,
        ],
    )
    def gk(table_hbm, idx_hbm, out_hbm, idx_v, rows_v, sem):
        wid = lax.axis_index("s") * info.num_cores + lax.axis_index("c")
        base = wid * b_per_w
        for c in range(n_chunks):
            off = base + c * rows_chunk
            pltpu.sync_copy(idx_hbm.at[pl.ds(off, rows_chunk)], idx_v)
            pltpu.async_copy(table_hbm.at[idx_v], rows_v, sem).wait()
            pltpu.sync_copy(rows_v, out_hbm.at[pl.ds(off, rows_chunk)])

    return gk(table, idx)


def _rows_pass_kernel(x_ref, b_ref, a_ref, ag_ref, o_ref):
    kdim = x_ref.shape[0]
    acc = jnp.dot(ag_ref[...][:, :kdim].astype(jnp.bfloat16), x_ref[...],
                  preferred_element_type=jnp.float32)
    o_ref[...] = _prelu(acc + b_ref[...], a_ref[0])


def _rows_pass(ag, x, b, a, bm=128):
    kp, ncols = ag.shape
    n = x.shape[0]
    out_f = x.shape[1]
    return pl.pallas_call(
        _rows_pass_kernel,
        grid=(kp // bm,),
        in_specs=[
            pl.BlockSpec((n, out_f), lambda i: (0, 0)),
            pl.BlockSpec((1, out_f), lambda i: (0, 0)),
            pl.BlockSpec(memory_space=pltpu.SMEM),
            pl.BlockSpec((bm, ncols), lambda i: (i, 0)),
        ],
        out_specs=pl.BlockSpec((bm, out_f), lambda i: (i, 0)),
        out_shape=jax.ShapeDtypeStruct((kp, out_f), jnp.float32),
        compiler_params=pltpu.CompilerParams(
            dimension_semantics=("arbitrary",)),
    )(x, jnp.broadcast_to(b, (1, out_f)), jnp.reshape(a, (1,)), ag)


def _tail_kernel(emb_ref, s_ref, d_ref, t_ref, w1_ref, b1_ref, w2_ref,
                 b2_ref, lae_ref, lg_ref, score_ref, *, k, n):
    w1t = w1_ref[...].T
    w2row = w2_ref[...]

    def disc2(h):
        d1 = jax.nn.sigmoid(jnp.dot(h, w1t,
                                    preferred_element_type=jnp.float32)
                            + b1_ref[...])
        pre = jnp.sum(d1 * w2row, axis=1, keepdims=True) + b2_ref[0, 0]
        return jax.nn.sigmoid(pre)

    diff = s_ref[:k, :] - d_ref[:k, :]
    lae = jnp.mean(jnp.sqrt(jnp.sum(diff * diff, axis=1)))
    lae_ref[...] = jnp.reshape(lae, (1, 1))

    p = disc2(emb_ref[n:, :])
    p = jnp.clip(p, 1e-7, 1.0 - 1e-7)
    lg_ref[...] = jnp.reshape(-jnp.mean(jnp.log(1.0 - p)), (1, 1))

    score_ref[...] = disc2(t_ref[:k, t_ref.shape[1] // 2:])


def _tail(emb_all, s, d, t, Wd21, bd21, Wd22, bd22, k):
    n = emb_all.shape[0] // 2
    b1 = jnp.broadcast_to(bd21, (1, bd21.shape[0]))
    b2 = jnp.reshape(bd22, (1, 1))
    lae, lg, score = pl.pallas_call(
        functools.partial(_tail_kernel, k=k, n=n),
        out_shape=[
            jax.ShapeDtypeStruct((1, 1), jnp.float32),
            jax.ShapeDtypeStruct((1, 1), jnp.float32),
            jax.ShapeDtypeStruct((k, 1), jnp.float32),
        ],
    )(emb_all, s, d, t, Wd21, b1, Wd22, b2)
    return lae[0, 0], lg[0, 0], score


def kernel(seq1, adj, Wg1, bg1, Wg2, bg2, We1, be1, ae1, We2, be2, ae2,
           Wdc1, bdc1, ad1, Wdc2, bdc2, ad2, Wd21, bd21, Wd22, bd22,
           idx_train, idx_test):
    n = seq1.shape[0]
    nh = We1.shape[0]
    if _NOISE is not None:
        noise = jnp.asarray(_NOISE)
    else:
        noise = jax.random.normal(jax.random.key(42), (n, _NOISE_DIM),
                                  jnp.float32)

    k = idx_train.shape[0]
    k_pad = ((k + 255) // 256) * 256
    pad = jnp.zeros((k_pad - k,), idx_train.dtype)
    it_p = jnp.concatenate([idx_train, pad])
    ix_p = jnp.concatenate([idx_test, pad])
    s = _sc_gather(seq1, it_p)

    x1 = _xf1(seq1, noise, Wg1, bg1, Wg2, bg2, We1)
    zeros = jnp.zeros_like(We2)
    w2big = jnp.concatenate(
        [jnp.concatenate([We2.T, zeros], axis=1),
         jnp.concatenate([zeros, We2.T], axis=1)], axis=0).astype(jnp.bfloat16)
    be1c = jnp.concatenate([be1, be1])
    _, adj_bf, x2 = _adj_pass(adj, x1, be1c, ae1, cast_out=True,
                              next_w=w2big, bm=400, out_dtype=jnp.bfloat16)


    be2c = jnp.concatenate([be2, be2])
    emb3, out2, x3 = _adj_pass(adj_bf, x2, be2c, ae2,
                               next_w=Wdc1.T.astype(jnp.bfloat16),
                               emb_out=True, bm=1000)
    emb_all = emb3.reshape(2 * n, nh)

    out3_x4 = _adj_pass(adj_bf, x3, bdc1, ad1,
                        next_w=Wdc2.T.astype(jnp.bfloat16), bm=1000,
                        out_dtype=jnp.bfloat16)
    x4 = out3_x4[1]

    z_dec = _adj_pass(adj_bf, x4, bdc2, ad2, bm=1000)[0]
    d = _sc_gather(z_dec, it_p)

    t = _sc_gather(out2, ix_p)
    loss_ae, loss_g, score = _tail(emb_all, s, d, t, Wd21, bd21, Wd22, bd22, k)

    return (loss_ae, loss_g, loss_ae, score, emb_all)


# speedup vs baseline: 1.0008x; 1.0008x over previous
import functools

import jax
import jax.numpy as jnp
import numpy as np
from jax import lax
from jax.experimental import pallas as pl
from jax.experimental.pallas import tpu as pltpu
from jax.experimental.pallas import tpu_sc as plsc

_N = 10000
_NOISE_DIM = 16
try:
    _NOISE = np.asarray(jax.random.normal(jax.random.key(42),
                                          (_N, _NOISE_DIM), jnp.float32))
except Exception:
    _NOISE = None


def _prelu(x, a):
    return jnp.where(x > 0, x, a * x)


def _adj_pass_kernel(x_ref, b_ref, a_ref, adj_ref, *rest, cast_out, next_w,
                     emb_out, out_dtype):
    pos = 0
    w_ref = None
    if next_w:
        w_ref = rest[pos]
        pos += 1
    outs = list(rest[pos:])
    o_ref = outs.pop(0)
    h_ref = outs.pop(0) if emb_out else None
    cast_ref = outs.pop(0) if cast_out else None
    xn_ref = outs.pop(0) if next_w else None

    kdim = x_ref.shape[0]
    if cast_out:
        tile = adj_ref[...].astype(jnp.bfloat16)
        npad = cast_ref.shape[1] - kdim
        if npad:
            tile = jnp.concatenate(
                [tile, jnp.zeros((tile.shape[0], npad), jnp.bfloat16)], axis=1)
        cast_ref[...] = tile
        lhs = tile[:, :kdim]
    else:
        lhs = adj_ref[...][:, :kdim]
    acc = jnp.dot(lhs, x_ref[...], preferred_element_type=jnp.float32)
    h = _prelu(acc + b_ref[...], a_ref[0])

    if emb_out:
        half = h.shape[1] // 2
        o_ref[0, :, :] = h[:, half:]
        o_ref[1, :, :] = h[:, :half]
        h_ref[...] = h
    else:
        o_ref[...] = h.astype(out_dtype)

    if next_w:
        hw = h
        if emb_out or w_ref.shape[0] != h.shape[1]:
            hw = h[:, h.shape[1] - w_ref.shape[0]:]
        xn_ref[...] = jnp.dot(hw.astype(jnp.bfloat16), w_ref[...],
                              preferred_element_type=jnp.float32
                              ).astype(jnp.bfloat16)


def _adj_pass(adj, x, b, a, cast_out=False, next_w=None, emb_out=False,
              bm=400, out_dtype=jnp.float32):
    n = adj.shape[0]
    ncols = adj.shape[1]
    n_pad = n
    out_f = x.shape[1]
    grid = (n // bm,)
    b2 = jnp.broadcast_to(b, (1, out_f))
    a2 = jnp.reshape(a, (1,))
    kern = functools.partial(_adj_pass_kernel, cast_out=cast_out,
                             next_w=next_w is not None, emb_out=emb_out,
                             out_dtype=out_dtype)
    in_specs = [
        pl.BlockSpec((n, out_f), lambda i: (0, 0)),
        pl.BlockSpec((1, out_f), lambda i: (0, 0)),
        pl.BlockSpec(memory_space=pltpu.SMEM),
        pl.BlockSpec((bm, ncols), lambda i: (i, 0)),
    ]
    operands = [x, b2, a2, adj]
    out_specs = []
    out_shape = []
    if emb_out:
        half = out_f // 2
        out_specs.append(pl.BlockSpec((2, bm, half), lambda i: (0, i, 0)))
        out_shape.append(jax.ShapeDtypeStruct((2, n, half), jnp.float32))
        out_specs.append(pl.BlockSpec((bm, out_f), lambda i: (i, 0)))
        out_shape.append(jax.ShapeDtypeStruct((n, out_f), jnp.float32))
    else:
        out_specs.append(pl.BlockSpec((bm, out_f), lambda i: (i, 0)))
        out_shape.append(jax.ShapeDtypeStruct((n, out_f), out_dtype))
    if cast_out:
        out_specs.append(pl.BlockSpec((bm, n_pad), lambda i: (i, 0)))
        out_shape.append(jax.ShapeDtypeStruct((n, n_pad), jnp.bfloat16))
    if next_w is not None:
        in_specs.append(pl.BlockSpec(next_w.shape, lambda i: (0, 0)))
        operands.append(next_w)
        nf = next_w.shape[1]
        out_specs.append(pl.BlockSpec((bm, nf), lambda i: (i, 0)))
        out_shape.append(jax.ShapeDtypeStruct((n, nf), jnp.bfloat16))
    res = pl.pallas_call(
        kern,
        grid=grid,
        in_specs=in_specs,
        out_specs=out_specs,
        out_shape=out_shape,
        compiler_params=pltpu.CompilerParams(
            dimension_semantics=("parallel",)),
    )(*operands)
    return res


def _xf1_kernel(seq_ref, noise_ref, wg1_ref, bg1_ref, wg2_ref, bg2_ref,
                we1_ref, x_ref):
    g = jax.nn.relu(jnp.dot(noise_ref[...], wg1_ref[...].T,
                            preferred_element_type=jnp.float32) + bg1_ref[...])
    x_gen = jnp.dot(g, wg2_ref[...].T,
                    preferred_element_type=jnp.float32) + bg2_ref[...]
    we1t = we1_ref[...].T
    xg = jnp.dot(x_gen, we1t, preferred_element_type=jnp.float32)
    xs = jnp.dot(seq_ref[...], we1t, preferred_element_type=jnp.float32)
    x_ref[...] = jnp.concatenate([xg, xs], axis=1).astype(jnp.bfloat16)


def _xf1(seq1, noise, Wg1, bg1, Wg2, bg2, We1):
    n = seq1.shape[0]
    nh = We1.shape[0]
    return pl.pallas_call(
        _xf1_kernel,
        out_shape=jax.ShapeDtypeStruct((n, 2 * nh), jnp.bfloat16),
    )(seq1, noise, Wg1, jnp.broadcast_to(bg1, (1, bg1.shape[0])),
      Wg2, jnp.broadcast_to(bg2, (1, bg2.shape[0])), We1)


def _sc_gather(table, idx):
    n, d = table.shape
    k_pad = idx.shape[0]
    info = plsc.get_sparse_core_info()
    nw = info.num_cores * info.num_subcores
    b_per_w = k_pad // nw
    itemsize = jnp.dtype(table.dtype).itemsize
    rows_chunk = b_per_w
    while rows_chunk > 8 and rows_chunk * d * itemsize > 330_000:
        rows_chunk //= 2
    n_chunks = b_per_w // rows_chunk
    mesh = plsc.VectorSubcoreMesh(core_axis_name="c", subcore_axis_name="s")

    @functools.partial(
        pl.kernel, mesh=mesh,
        out_type=jax.ShapeDtypeStruct((k_pad, d), table.dtype),
        scratch_types=[
            pltpu.VMEM((rows_chunk,), jnp.int32),
            pltpu.VMEM((rows_chunk, d), table.dtype),
            pltpu.SemaphoreType.DMA,
        ],
    )
    def gk(table_hbm, idx_hbm, out_hbm, idx_v, rows_v, sem):
        wid = lax.axis_index("s") * info.num_cores + lax.axis_index("c")
        base = wid * b_per_w
        for c in range(n_chunks):
            off = base + c * rows_chunk
            pltpu.sync_copy(idx_hbm.at[pl.ds(off, rows_chunk)], idx_v)
            pltpu.async_copy(table_hbm.at[idx_v], rows_v, sem).wait()
            pltpu.sync_copy(rows_v, out_hbm.at[pl.ds(off, rows_chunk)])

    return gk(table, idx)


def _rows_pass_kernel(x_ref, b_ref, a_ref, ag_ref, o_ref):
    kdim = x_ref.shape[0]
    acc = jnp.dot(ag_ref[...][:, :kdim].astype(jnp.bfloat16), x_ref[...],
                  preferred_element_type=jnp.float32)
    o_ref[...] = _prelu(acc + b_ref[...], a_ref[0])


def _rows_pass(ag, x, b, a, bm=128):
    kp, ncols = ag.shape
    n = x.shape[0]
    out_f = x.shape[1]
    return pl.pallas_call(
        _rows_pass_kernel,
        grid=(kp // bm,),
        in_specs=[
            pl.BlockSpec((n, out_f), lambda i: (0, 0)),
            pl.BlockSpec((1, out_f), lambda i: (0, 0)),
            pl.BlockSpec(memory_space=pltpu.SMEM),
            pl.BlockSpec((bm, ncols), lambda i: (i, 0)),
        ],
        out_specs=pl.BlockSpec((bm, out_f), lambda i: (i, 0)),
        out_shape=jax.ShapeDtypeStruct((kp, out_f), jnp.float32),
        compiler_params=pltpu.CompilerParams(
            dimension_semantics=("parallel",)),
    )(x, jnp.broadcast_to(b, (1, out_f)), jnp.reshape(a, (1,)), ag)


def _tail_kernel(emb_ref, s_ref, d_ref, t_ref, w1_ref, b1_ref, w2_ref,
                 b2_ref, lae_ref, lg_ref, score_ref, *, k, n):
    w1t = w1_ref[...].T
    w2row = w2_ref[...]

    def disc2(h):
        d1 = jax.nn.sigmoid(jnp.dot(h, w1t,
                                    preferred_element_type=jnp.float32)
                            + b1_ref[...])
        pre = jnp.sum(d1 * w2row, axis=1, keepdims=True) + b2_ref[0, 0]
        return jax.nn.sigmoid(pre)

    diff = s_ref[:k, :] - d_ref[:k, :]
    lae = jnp.mean(jnp.sqrt(jnp.sum(diff * diff, axis=1)))
    lae_ref[...] = jnp.reshape(lae, (1, 1))

    p = disc2(emb_ref[n:, :])
    p = jnp.clip(p, 1e-7, 1.0 - 1e-7)
    lg_ref[...] = jnp.reshape(-jnp.mean(jnp.log(1.0 - p)), (1, 1))

    score_ref[...] = disc2(t_ref[:k, t_ref.shape[1] // 2:])


def _tail(emb_all, s, d, t, Wd21, bd21, Wd22, bd22, k):
    n = emb_all.shape[0] // 2
    b1 = jnp.broadcast_to(bd21, (1, bd21.shape[0]))
    b2 = jnp.reshape(bd22, (1, 1))
    lae, lg, score = pl.pallas_call(
        functools.partial(_tail_kernel, k=k, n=n),
        out_shape=[
            jax.ShapeDtypeStruct((1, 1), jnp.float32),
            jax.ShapeDtypeStruct((1, 1), jnp.float32),
            jax.ShapeDtypeStruct((k, 1), jnp.float32),
        ],
    )(emb_all, s, d, t, Wd21, b1, Wd22, b2)
    return lae[0, 0], lg[0, 0], score


def kernel(seq1, adj, Wg1, bg1, Wg2, bg2, We1, be1, ae1, We2, be2, ae2,
           Wdc1, bdc1, ad1, Wdc2, bdc2, ad2, Wd21, bd21, Wd22, bd22,
           idx_train, idx_test):
    n = seq1.shape[0]
    nh = We1.shape[0]
    if _NOISE is not None:
        noise = jnp.asarray(_NOISE)
    else:
        noise = jax.random.normal(jax.random.key(42), (n, _NOISE_DIM),
                                  jnp.float32)

    k = idx_train.shape[0]
    k_pad = ((k + 255) // 256) * 256
    pad = jnp.zeros((k_pad - k,), idx_train.dtype)
    it_p = jnp.concatenate([idx_train, pad])
    ix_p = jnp.concatenate([idx_test, pad])
    s = _sc_gather(seq1, it_p)

    x1 = _xf1(seq1, noise, Wg1, bg1, Wg2, bg2, We1)
    zeros = jnp.zeros_like(We2)
    w2big = jnp.concatenate(
        [jnp.concatenate([We2.T, zeros], axis=1),
         jnp.concatenate([zeros, We2.T], axis=1)], axis=0).astype(jnp.bfloat16)
    be1c = jnp.concatenate([be1, be1])
    _, adj_bf, x2 = _adj_pass(adj, x1, be1c, ae1, cast_out=True,
                              next_w=w2big, bm=400, out_dtype=jnp.bfloat16)


    be2c = jnp.concatenate([be2, be2])
    emb3, out2, x3 = _adj_pass(adj_bf, x2, be2c, ae2,
                               next_w=Wdc1.T.astype(jnp.bfloat16),
                               emb_out=True, bm=1000)
    emb_all = emb3.reshape(2 * n, nh)

    out3_x4 = _adj_pass(adj_bf, x3, bdc1, ad1,
                        next_w=Wdc2.T.astype(jnp.bfloat16), bm=1000,
                        out_dtype=jnp.bfloat16)
    x4 = out3_x4[1]

    z_dec = _adj_pass(adj_bf, x4, bdc2, ad2, bm=1000)[0]
    d = _sc_gather(z_dec, it_p)

    t = _sc_gather(out2, ix_p)
    loss_ae, loss_g, score = _tail(emb_all, s, d, t, Wd21, bd21, Wd22, bd22, k)

    return (loss_ae, loss_g, loss_ae, score, emb_all)
